# baseline (speedup 1.0000x reference)
import functools

import jax
import jax.numpy as jnp
from jax import lax
from jax.experimental import pallas as pl
from jax.experimental.pallas import tpu as pltpu
from jax.experimental.pallas import tpu_sc as plsc

N = 50000
D = 256
G = 128
L = 16
NC = 2
NS = 16
NW = NC * NS

BLK = 128
NBLK = N // BLK
TAIL = N - NBLK * BLK
EXTRA = NBLK - NW * (NBLK // NW)

B1 = 96
NB1 = N // B1
E1 = NB1 - NW * (NB1 // NW)

_mesh = functools.partial(
    plsc.VectorSubcoreMesh,
    core_axis_name="c",
    subcore_axis_name="s",
    num_cores=NC,
    num_subcores=NS,
)


def _worker_id():
    return lax.axis_index("c") * NS + lax.axis_index("s")


@functools.partial(
    pl.kernel,
    out_type=(
        jax.ShapeDtypeStruct((G, NW * D), jnp.float32),
        jax.ShapeDtypeStruct((NW, G), jnp.float32),
    ),
    mesh=_mesh(),
    scratch_types=(
        pltpu.VMEM((B1, D), jnp.float32),
        pltpu.VMEM((B1, D), jnp.float32),
        pltpu.VMEM((B1, D), jnp.float32),
        pltpu.VMEM((B1,), jnp.int32),
        pltpu.VMEM((B1,), jnp.int32),
        pltpu.VMEM((B1,), jnp.int32),
        pltpu.VMEM((TAIL,), jnp.int32),
        pltpu.VMEM((B1,), jnp.int32),
        pltpu.VMEM((B1,), jnp.int32),
        pltpu.VMEM((G,), jnp.float32),
        pltpu.VMEM((G, D), jnp.float32),
        pltpu.SemaphoreType.DMA,
        pltpu.SemaphoreType.DMA,
        pltpu.SemaphoreType.DMA,
    ),
    compiler_params=pltpu.CompilerParams(needs_layout_passes=False),
)
def _k1(x_hbm, batch_hbm, psums_hbm, pcounts_hbm, xb0, xb1, xb2, ib0, ib1,
        ib2, idxtail, gi1, gi2, cnt_v, acc_v, sem0, sem1, sem2):
    c = lax.axis_index("c")
    s = lax.axis_index("s")
    w = c * NS + s
    zero16 = jnp.zeros((L,), jnp.float32)
    ones16 = jnp.ones((L,), jnp.float32)

    for i in range(G // L):
        cnt_v[pl.ds(i * L, L)] = zero16

    def zrow(r, carry):
        for i in range(D // L):
            acc_v[r, pl.ds(i * L, L)] = zero16
        return carry

    lax.fori_loop(0, G, zrow, 0)

    nblk = jnp.where(w < E1, NB1 // NW + 1, NB1 // NW)

    def issue(j, xbuf, ibuf, sem):
        base = (w + j * NW) * B1
        pltpu.async_copy(batch_hbm.at[pl.ds(base, B1)], ibuf, sem)
        pltpu.async_copy(x_hbm.at[pl.ds(base, B1), :], xbuf, sem)

    def drain(xbuf, ibuf, sem):
        pltpu.make_async_copy(batch_hbm.at[pl.ds(0, B1)], ibuf, sem).wait()
        pltpu.make_async_copy(x_hbm.at[pl.ds(0, B1), :], xbuf, sem).wait()

    iota16 = lax.iota(jnp.int32, L)

    def make_issue_s(gidx):
        def issue_s(j, xbuf, ibuf, sem):
            base = (w + j * NW) * B1
            pltpu.async_copy(batch_hbm.at[pl.ds(base, B1)], ibuf, sem)
            for i in range(B1 // L):
                gidx[pl.ds(i * L, L)] = iota16 + (base + i * L)
            pltpu.async_copy(x_hbm.at[gidx], xbuf, sem)
        return issue_s

    def make_drain_s(gidx):
        def drain_s(xbuf, ibuf, sem):
            pltpu.make_async_copy(batch_hbm.at[pl.ds(0, B1)], ibuf,
                                  sem).wait()
            pltpu.make_async_copy(x_hbm.at[gidx], xbuf, sem).wait()
        return drain_s

    def process(nrows, xbuf, ibuf):
        g0 = plsc.load_gather(ibuf, [jnp.full((L,), 0, jnp.int32)])
        gl = plsc.load_gather(ibuf, [jnp.full((L,), nrows - 1, jnp.int32)])
        pure = g0[0] == gl[0]

        @pl.when(pure)
        def _fast():
            def chunk_body(cb, carry):
                regs = carry
                out = list(regs)
                for rr in range(L):
                    r = cb * L + rr
                    for i in range(D // L):
                        out[i] = out[i] + xbuf[r, pl.ds(i * L, L)]
                return tuple(out)

            init = tuple(zero16 for _ in range(D // L))
            final = lax.fori_loop(0, nrows // L, chunk_body, init)
            row = g0[0]
            for i in range(D // L):
                sl = pl.ds(i * L, L)
                acc_v[row, sl] = acc_v[row, sl] + final[i]

        @pl.when(jnp.logical_not(pure))
        def _slow():
            def row_body(r, carry):
                pg = carry[0]
                regs = carry[1:]
                g16 = plsc.load_gather(ibuf, [jnp.full((L,), r, jnp.int32)])
                changed = g16[0] != pg[0]

                @pl.when(changed)
                def _flush():
                    row = pg[0]
                    for i in range(D // L):
                        sl = pl.ds(i * L, L)
                        acc_v[row, sl] = acc_v[row, sl] + regs[i]

                new_regs = []
                for i in range(D // L):
                    xv = xbuf[r, pl.ds(i * L, L)]
                    new_regs.append(jnp.where(changed, xv, regs[i] + xv))
                return (g16, *new_regs)

            init = (g0,) + tuple(zero16 for _ in range(D // L))
            final = lax.fori_loop(0, nrows, row_body, init)
            pgf = final[0]
            rowf = pgf[0]
            for i in range(D // L):
                sl = pl.ds(i * L, L)
                acc_v[rowf, sl] = acc_v[rowf, sl] + final[1 + i]

        for i in range(nrows // L):
            iv = ibuf[pl.ds(i * L, L)]
            plsc.addupdate_scatter(cnt_v, [iv], ones16)

    bufs = ((xb0, ib0, sem0, issue, drain),
            (xb1, ib1, sem1, make_issue_s(gi1), make_drain_s(gi1)),
            (xb2, ib2, sem2, make_issue_s(gi2), make_drain_s(gi2)))
    for b in range(3):
        xbuf, ibuf, sem, iss, _ = bufs[b]
        iss(b, xbuf, ibuf, sem)

    def block_body(t, carry):
        j3 = t * 3
        for b in range(3):
            xbuf, ibuf, sem, iss, drn = bufs[b]

            @pl.when(j3 + b < nblk)
            def _step(xbuf=xbuf, ibuf=ibuf, sem=sem, iss=iss, drn=drn, b=b):
                drn(xbuf, ibuf, sem)
                process(B1, xbuf, ibuf)

                @pl.when(j3 + b + 3 < nblk)
                def _pf():
                    iss(j3 + b + 3, xbuf, ibuf, sem)

        return carry

    lax.fori_loop(0, (nblk + 2) // 3, block_body, 0)

    @pl.when(w == NW - 1)
    def _tail():
        pltpu.sync_copy(batch_hbm.at[pl.ds(N - TAIL, TAIL)], idxtail)
        pltpu.sync_copy(x_hbm.at[pl.ds(N - TAIL, TAIL), :],
                        xb0.at[pl.ds(0, TAIL), :])

        def trow(r, carry):
            g16 = plsc.load_gather(idxtail, [jnp.full((L,), r, jnp.int32)])
            row = g16[0]
            for i in range(D // L):
                sl = pl.ds(i * L, L)
                acc_v[row, sl] = acc_v[row, sl] + xb0[r, sl]
            return carry

        lax.fori_loop(0, TAIL, trow, 0)
        for i in range(TAIL // L):
            iv = idxtail[pl.ds(i * L, L)]
            plsc.addupdate_scatter(cnt_v, [iv], ones16)

    pltpu.sync_copy(acc_v, psums_hbm.at[:, pl.ds(w * D, D)])
    pltpu.sync_copy(cnt_v, pcounts_hbm.at[w])


@functools.partial(
    pl.kernel,
    out_type=(
        jax.ShapeDtypeStruct((G, D), jnp.float32),
        jax.ShapeDtypeStruct((N,), jnp.float32),
    ),
    mesh=_mesh(),
    scratch_types=(
        pltpu.VMEM((NW, G), jnp.float32),
        pltpu.VMEM((4, NW * D), jnp.float32),
        pltpu.VMEM((4, D), jnp.float32),
        pltpu.VMEM((G,), jnp.float32),
        pltpu.VMEM((BLK,), jnp.int32),
        pltpu.VMEM((BLK,), jnp.int32),
        pltpu.VMEM((TAIL,), jnp.int32),
        pltpu.VMEM((BLK,), jnp.float32),
        pltpu.VMEM((BLK,), jnp.float32),
        pltpu.SemaphoreType.DMA,
        pltpu.SemaphoreType.DMA,
        pltpu.SemaphoreType.DMA,
        pltpu.SemaphoreType.DMA,
        pltpu.SemaphoreType.DMA,
        pltpu.SemaphoreType.DMA,
        pltpu.SemaphoreType.DMA,
    ),
    compiler_params=pltpu.CompilerParams(needs_layout_passes=False),
)
def _k2(batch_hbm, psums_hbm, pcounts_hbm, emb_hbm, scores_hbm,
        pc_v, pp_v, eout_v, inv_v, idx0, idx1, idxtail, sv0, sv1,
        sem_pc, sem_pp, semi0, semi1, semo0, semo1, sem_emb):
    w = _worker_id()
    r0 = w * (G // NW)
    nblk = jnp.where(w < EXTRA, NBLK // NW + 1, NBLK // NW)

    pltpu.async_copy(pcounts_hbm, pc_v, sem_pc)
    pltpu.async_copy(psums_hbm.at[pl.ds(r0, 4), :], pp_v, sem_pp)
    pltpu.async_copy(batch_hbm.at[pl.ds(w * BLK, BLK)], idx0, semi0)
    pltpu.async_copy(batch_hbm.at[pl.ds((w + NW) * BLK, BLK)], idx1, semi1)

    pltpu.make_async_copy(pcounts_hbm, pc_v, sem_pc).wait()
    for i in range(G // L):
        acc = jnp.zeros((L,), jnp.float32)
        for t in range(NW):
            acc = acc + pc_v[t, pl.ds(i * L, L)]
        inv_v[pl.ds(i * L, L)] = 1.0 / jnp.maximum(acc, 1.0)

    pltpu.make_async_copy(psums_hbm.at[pl.ds(r0, 4), :], pp_v, sem_pp).wait()
    for r in range(4):
        ridx = jnp.full((L,), r0 + r, jnp.int32)
        ivs = plsc.load_gather(inv_v, [ridx])

        def red_body(t, carry):
            for i in range(D // L):
                sl = pl.ds(i * L, L)
                prev = jnp.where(t == 0, jnp.zeros((L,), jnp.float32),
                                 eout_v[r, sl])
                eout_v[r, sl] = prev + pp_v[r, pl.ds(t * D + i * L, L)]
            return carry

        lax.fori_loop(0, NW, red_body, 0)
        for i in range(D // L):
            sl = pl.ds(i * L, L)
            eout_v[r, sl] = eout_v[r, sl] * ivs
    pltpu.async_copy(eout_v, emb_hbm.at[pl.ds(r0, 4), :], sem_emb)

    def step(j, ibuf, sbuf, semi, semo):
        base = (w + j * NW) * BLK
        pltpu.make_async_copy(batch_hbm.at[pl.ds(0, BLK)], ibuf, semi).wait()

        @pl.when(j >= 2)
        def _reuse():
            pltpu.make_async_copy(sbuf, scores_hbm.at[pl.ds(0, BLK)],
                                  semo).wait()

        for i in range(BLK // L):
            iv = ibuf[pl.ds(i * L, L)]
            sbuf[pl.ds(i * L, L)] = plsc.load_gather(inv_v, [iv])

        @pl.when(j + 2 < nblk)
        def _pf():
            nbase = (w + (j + 2) * NW) * BLK
            pltpu.async_copy(batch_hbm.at[pl.ds(nbase, BLK)], ibuf, semi)

        pltpu.async_copy(sbuf, scores_hbm.at[pl.ds(base, BLK)], semo)

    def block_body(j, carry):
        even = lax.rem(j, 2) == 0

        @pl.when(even)
        def _even():
            step(j, idx0, sv0, semi0, semo0)

        @pl.when(jnp.logical_not(even))
        def _odd():
            step(j, idx1, sv1, semi1, semo1)

        return carry

    lax.fori_loop(0, nblk, block_body, 0)

    pltpu.make_async_copy(sv0, scores_hbm.at[pl.ds(0, BLK)], semo0).wait()
    pltpu.make_async_copy(sv1, scores_hbm.at[pl.ds(0, BLK)], semo1).wait()

    @pl.when(w == NW - 1)
    def _tail():
        pltpu.sync_copy(batch_hbm.at[pl.ds(N - TAIL, TAIL)], idxtail)
        for i in range(TAIL // L):
            iv = idxtail[pl.ds(i * L, L)]
            sv0[pl.ds(i * L, L)] = plsc.load_gather(inv_v, [iv])
        pltpu.sync_copy(sv0.at[pl.ds(0, TAIL)],
                        scores_hbm.at[pl.ds(N - TAIL, TAIL)])

    pltpu.make_async_copy(eout_v, emb_hbm.at[pl.ds(r0, 4), :], sem_emb).wait()


def kernel(x, batch):
    psums, pcounts = _k1(x, batch)
    emb, scores = _k2(batch, psums, pcounts)
    return emb, scores


# speedup vs baseline: 1.0298x; 1.0298x over previous
import functools

import jax
import jax.numpy as jnp
from jax import lax
from jax.experimental import pallas as pl
from jax.experimental.pallas import tpu as pltpu
from jax.experimental.pallas import tpu_sc as plsc

N = 50000
D = 256
G = 128
L = 16
NC = 2
NS = 16
NW = NC * NS

BLK = 128
NBLK = N // BLK
TAIL = N - NBLK * BLK
EXTRA = NBLK - NW * (NBLK // NW)

B1 = 96
NB1 = N // B1
E1 = NB1 - NW * (NB1 // NW)

SB = 1568
SLAST = N - (NW - 1) * SB

_mesh = functools.partial(
    plsc.VectorSubcoreMesh,
    core_axis_name="c",
    subcore_axis_name="s",
    num_cores=NC,
    num_subcores=NS,
)


def _worker_id():
    return lax.axis_index("c") * NS + lax.axis_index("s")


@functools.partial(
    pl.kernel,
    out_type=(
        jax.ShapeDtypeStruct((G, NW * D), jnp.float32),
        jax.ShapeDtypeStruct((NW, G), jnp.float32),
    ),
    mesh=_mesh(),
    scratch_types=(
        pltpu.VMEM((B1, D), jnp.float32),
        pltpu.VMEM((B1, D), jnp.float32),
        pltpu.VMEM((B1, D), jnp.float32),
        pltpu.VMEM((B1,), jnp.int32),
        pltpu.VMEM((B1,), jnp.int32),
        pltpu.VMEM((B1,), jnp.int32),
        pltpu.VMEM((TAIL,), jnp.int32),
        pltpu.VMEM((B1,), jnp.int32),
        pltpu.VMEM((B1,), jnp.int32),
        pltpu.VMEM((G,), jnp.float32),
        pltpu.VMEM((G, D), jnp.float32),
        pltpu.SemaphoreType.DMA,
        pltpu.SemaphoreType.DMA,
        pltpu.SemaphoreType.DMA,
    ),
    compiler_params=pltpu.CompilerParams(needs_layout_passes=False),
)
def _k1(x_hbm, batch_hbm, psums_hbm, pcounts_hbm, xb0, xb1, xb2, ib0, ib1,
        ib2, idxtail, gi1, gi2, cnt_v, acc_v, sem0, sem1, sem2):
    c = lax.axis_index("c")
    s = lax.axis_index("s")
    w = c * NS + s
    zero16 = jnp.zeros((L,), jnp.float32)
    ones16 = jnp.ones((L,), jnp.float32)

    for i in range(G // L):
        cnt_v[pl.ds(i * L, L)] = zero16

    def zrow(r, carry):
        for i in range(D // L):
            acc_v[r, pl.ds(i * L, L)] = zero16
        return carry

    lax.fori_loop(0, G, zrow, 0)

    nblk = jnp.where(w < E1, NB1 // NW + 1, NB1 // NW)

    def issue(j, xbuf, ibuf, sem):
        base = (w + j * NW) * B1
        pltpu.async_copy(batch_hbm.at[pl.ds(base, B1)], ibuf, sem)
        pltpu.async_copy(x_hbm.at[pl.ds(base, B1), :], xbuf, sem)

    def drain(xbuf, ibuf, sem):
        pltpu.make_async_copy(batch_hbm.at[pl.ds(0, B1)], ibuf, sem).wait()
        pltpu.make_async_copy(x_hbm.at[pl.ds(0, B1), :], xbuf, sem).wait()

    iota16 = lax.iota(jnp.int32, L)

    def make_issue_s(gidx):
        def issue_s(j, xbuf, ibuf, sem):
            base = (w + j * NW) * B1
            pltpu.async_copy(batch_hbm.at[pl.ds(base, B1)], ibuf, sem)
            for i in range(B1 // L):
                gidx[pl.ds(i * L, L)] = iota16 + (base + i * L)
            pltpu.async_copy(x_hbm.at[gidx], xbuf, sem)
        return issue_s

    def make_drain_s(gidx):
        def drain_s(xbuf, ibuf, sem):
            pltpu.make_async_copy(batch_hbm.at[pl.ds(0, B1)], ibuf,
                                  sem).wait()
            pltpu.make_async_copy(x_hbm.at[gidx], xbuf, sem).wait()
        return drain_s

    def process(nrows, xbuf, ibuf):
        g0 = plsc.load_gather(ibuf, [jnp.full((L,), 0, jnp.int32)])
        gl = plsc.load_gather(ibuf, [jnp.full((L,), nrows - 1, jnp.int32)])
        pure = g0[0] == gl[0]

        @pl.when(pure)
        def _fast():
            def chunk_body(cb, carry):
                regs = carry
                out = list(regs)
                for rr in range(L):
                    r = cb * L + rr
                    for i in range(D // L):
                        out[i] = out[i] + xbuf[r, pl.ds(i * L, L)]
                return tuple(out)

            init = tuple(zero16 for _ in range(D // L))
            final = lax.fori_loop(0, nrows // L, chunk_body, init)
            row = g0[0]
            for i in range(D // L):
                sl = pl.ds(i * L, L)
                acc_v[row, sl] = acc_v[row, sl] + final[i]

        @pl.when(jnp.logical_not(pure))
        def _slow():
            def row_body(r, carry):
                pg = carry[0]
                regs = carry[1:]
                g16 = plsc.load_gather(ibuf, [jnp.full((L,), r, jnp.int32)])
                changed = g16[0] != pg[0]

                @pl.when(changed)
                def _flush():
                    row = pg[0]
                    for i in range(D // L):
                        sl = pl.ds(i * L, L)
                        acc_v[row, sl] = acc_v[row, sl] + regs[i]

                new_regs = []
                for i in range(D // L):
                    xv = xbuf[r, pl.ds(i * L, L)]
                    new_regs.append(jnp.where(changed, xv, regs[i] + xv))
                return (g16, *new_regs)

            init = (g0,) + tuple(zero16 for _ in range(D // L))
            final = lax.fori_loop(0, nrows, row_body, init)
            pgf = final[0]
            rowf = pgf[0]
            for i in range(D // L):
                sl = pl.ds(i * L, L)
                acc_v[rowf, sl] = acc_v[rowf, sl] + final[1 + i]

        for i in range(nrows // L):
            iv = ibuf[pl.ds(i * L, L)]
            plsc.addupdate_scatter(cnt_v, [iv], ones16)

    bufs = ((xb0, ib0, sem0, issue, drain),
            (xb1, ib1, sem1, make_issue_s(gi1), make_drain_s(gi1)),
            (xb2, ib2, sem2, issue, drain))
    for b in range(3):
        xbuf, ibuf, sem, iss, _ = bufs[b]
        iss(b, xbuf, ibuf, sem)

    def block_body(t, carry):
        j3 = t * 3
        for b in range(3):
            xbuf, ibuf, sem, iss, drn = bufs[b]

            @pl.when(j3 + b < nblk)
            def _step(xbuf=xbuf, ibuf=ibuf, sem=sem, iss=iss, drn=drn, b=b):
                drn(xbuf, ibuf, sem)
                process(B1, xbuf, ibuf)

                @pl.when(j3 + b + 3 < nblk)
                def _pf():
                    iss(j3 + b + 3, xbuf, ibuf, sem)

        return carry

    lax.fori_loop(0, (nblk + 2) // 3, block_body, 0)

    @pl.when(w == NW - 1)
    def _tail():
        pltpu.sync_copy(batch_hbm.at[pl.ds(N - TAIL, TAIL)], idxtail)
        pltpu.sync_copy(x_hbm.at[pl.ds(N - TAIL, TAIL), :],
                        xb0.at[pl.ds(0, TAIL), :])

        def trow(r, carry):
            g16 = plsc.load_gather(idxtail, [jnp.full((L,), r, jnp.int32)])
            row = g16[0]
            for i in range(D // L):
                sl = pl.ds(i * L, L)
                acc_v[row, sl] = acc_v[row, sl] + xb0[r, sl]
            return carry

        lax.fori_loop(0, TAIL, trow, 0)
        for i in range(TAIL // L):
            iv = idxtail[pl.ds(i * L, L)]
            plsc.addupdate_scatter(cnt_v, [iv], ones16)

    pltpu.sync_copy(acc_v, psums_hbm.at[:, pl.ds(w * D, D)])
    pltpu.sync_copy(cnt_v, pcounts_hbm.at[w])


@functools.partial(
    pl.kernel,
    out_type=(
        jax.ShapeDtypeStruct((G, D), jnp.float32),
        jax.ShapeDtypeStruct((N,), jnp.float32),
    ),
    mesh=_mesh(),
    scratch_types=(
        pltpu.VMEM((NW, G), jnp.float32),
        pltpu.VMEM((4, NW * D), jnp.float32),
        pltpu.VMEM((4, D), jnp.float32),
        pltpu.VMEM((G,), jnp.float32),
        pltpu.VMEM((SB,), jnp.int32),
        pltpu.VMEM((SB,), jnp.float32),
        pltpu.SemaphoreType.DMA,
        pltpu.SemaphoreType.DMA,
        pltpu.SemaphoreType.DMA,
        pltpu.SemaphoreType.DMA,
    ),
    compiler_params=pltpu.CompilerParams(needs_layout_passes=False),
)
def _k2(batch_hbm, psums_hbm, pcounts_hbm, emb_hbm, scores_hbm,
        pc_v, pp_v, eout_v, inv_v, idxs, svs,
        sem_pc, sem_pp, sem_idx, sem_emb):
    w = _worker_id()
    r0 = w * (G // NW)
    start = w * SB

    pltpu.async_copy(pcounts_hbm, pc_v, sem_pc)
    pltpu.async_copy(psums_hbm.at[pl.ds(r0, 4), :], pp_v, sem_pp)

    @pl.when(w < NW - 1)
    def _idx_full():
        pltpu.async_copy(batch_hbm.at[pl.ds(start, SB)], idxs, sem_idx)

    @pl.when(w == NW - 1)
    def _idx_last():
        pltpu.async_copy(batch_hbm.at[pl.ds((NW - 1) * SB, SLAST)],
                         idxs.at[pl.ds(0, SLAST)], sem_idx)

    pltpu.make_async_copy(pcounts_hbm, pc_v, sem_pc).wait()
    for i in range(G // L):
        acc = jnp.zeros((L,), jnp.float32)
        for t in range(NW):
            acc = acc + pc_v[t, pl.ds(i * L, L)]
        inv_v[pl.ds(i * L, L)] = 1.0 / jnp.maximum(acc, 1.0)

    pltpu.make_async_copy(psums_hbm.at[pl.ds(r0, 4), :], pp_v, sem_pp).wait()
    for r in range(4):
        ridx = jnp.full((L,), r0 + r, jnp.int32)
        ivs = plsc.load_gather(inv_v, [ridx])

        def red_body(t, carry):
            for i in range(D // L):
                sl = pl.ds(i * L, L)
                prev = jnp.where(t == 0, jnp.zeros((L,), jnp.float32),
                                 eout_v[r, sl])
                eout_v[r, sl] = prev + pp_v[r, pl.ds(t * D + i * L, L)]
            return carry

        lax.fori_loop(0, NW, red_body, 0)
        for i in range(D // L):
            sl = pl.ds(i * L, L)
            eout_v[r, sl] = eout_v[r, sl] * ivs
    pltpu.async_copy(eout_v, emb_hbm.at[pl.ds(r0, 4), :], sem_emb)

    nch = jnp.where(w < NW - 1, SB // L, SLAST // L)

    @pl.when(w < NW - 1)
    def _idx_wait_full():
        pltpu.make_async_copy(batch_hbm.at[pl.ds(0, SB)], idxs,
                              sem_idx).wait()

    @pl.when(w == NW - 1)
    def _idx_wait_last():
        pltpu.make_async_copy(batch_hbm.at[pl.ds(0, SLAST)],
                              idxs.at[pl.ds(0, SLAST)], sem_idx).wait()

    def gbody(i, carry):
        iv = idxs[pl.ds(i * L, L)]
        svs[pl.ds(i * L, L)] = plsc.load_gather(inv_v, [iv])
        return carry

    lax.fori_loop(0, nch, gbody, 0)

    @pl.when(w < NW - 1)
    def _out_full():
        pltpu.sync_copy(svs, scores_hbm.at[pl.ds(start, SB)])

    @pl.when(w == NW - 1)
    def _out_last():
        pltpu.sync_copy(svs.at[pl.ds(0, SLAST)],
                        scores_hbm.at[pl.ds((NW - 1) * SB, SLAST)])

    pltpu.make_async_copy(eout_v, emb_hbm.at[pl.ds(r0, 4), :], sem_emb).wait()


def kernel(x, batch):
    psums, pcounts = _k1(x, batch)
    emb, scores = _k2(batch, psums, pcounts)
    return emb, scores
